# baseline (speedup 1.0000x reference)
import functools

import jax
import jax.numpy as jnp
from jax import lax
from jax.experimental import pallas as pl
from jax.experimental.pallas import tpu as pltpu
from jax.experimental.pallas import tpu_sc as plsc

N = 10000
E = 320000
D = 128

NC = 2
NS = 16
NW = NC * NS

B = 100
CHUNKS = 100
ROWS_PER_TILE = N // NS


def _zero_fill(ref, nrows, ncols):
  @pl.loop(0, nrows)
  def _(r):
    for k in range(ncols // 16):
      ref[r, pl.ds(16 * k, 16)] = jnp.zeros((16,), jnp.float32)


_MESH = plsc.VectorSubcoreMesh(core_axis_name="c", subcore_axis_name="s")
_SC_PARAMS = pltpu.CompilerParams(use_tc_tiling_on_sc=False)

GROUPS = 5
GCH = CHUNKS // GROUPS


def _make_sc_agg(with_counts):
  def body(feats, edges, *rest):
    if with_counts:
      (acc_out, cnt_out, src_idx, dst_idx, rows0, rows1,
       sem0, sem1, semi, acc_sh, ones_v, cnt_sh) = rest
    else:
      (acc_out, src_idx, dst_idx, rows0, rows1,
       sem0, sem1, semi, acc_sh) = rest
    cid = lax.axis_index("c")
    sid = lax.axis_index("s")
    blk = cid * NS + sid
    base = sid * ROWS_PER_TILE

    _zero_fill(rows0, B, D)
    nfull = ROWS_PER_TILE // B
    tail = ROWS_PER_TILE - nfull * B
    for k in range(nfull):
      pltpu.sync_copy(rows0, acc_sh.at[pl.ds(base + k * B, B)])
    if tail:
      pltpu.sync_copy(rows0.at[pl.ds(0, tail)],
                      acc_sh.at[pl.ds(base + nfull * B, tail)])
    if with_counts:
      _zero_fill(ones_v, B, 16)
      for k in range(nfull):
        pltpu.sync_copy(ones_v, cnt_sh.at[pl.ds(base + k * B, B)])
      if tail:
        pltpu.sync_copy(ones_v.at[pl.ds(0, tail)],
                        cnt_sh.at[pl.ds(base + nfull * B, tail)])
      @pl.loop(0, B)
      def _(r):
        ones_v[r, :] = jnp.ones((16,), jnp.float32)

    pltpu.sync_copy(edges.at[0, blk, pl.ds(0, GCH)], src_idx.at[0])
    pltpu.sync_copy(edges.at[1, blk, pl.ds(0, GCH)], dst_idx.at[0])

    plsc.subcore_barrier()

    def start(p, t, buf, sem):
      pltpu.async_copy(feats.at[src_idx.at[p, t]], buf, sem)

    def wait(buf, sem):
      pltpu.make_async_copy(feats.at[pl.ds(0, B)], buf, sem).wait()

    def scat(p, t, buf):
      pltpu.sync_copy(buf, acc_sh.at[dst_idx.at[p, t]], add=True)
      if with_counts:
        pltpu.sync_copy(ones_v, cnt_sh.at[dst_idx.at[p, t]], add=True)

    start(0, 0, rows0, sem0)
    start(0, 1, rows1, sem1)

    for g in range(GROUPS):
      p = g % 2
      q = 1 - p
      if g < GROUPS - 1:
        pltpu.async_copy(edges.at[0, blk, pl.ds((g + 1) * GCH, GCH)],
                         src_idx.at[q], semi)
        pltpu.async_copy(edges.at[1, blk, pl.ds((g + 1) * GCH, GCH)],
                         dst_idx.at[q], semi)

      @pl.loop(0, GCH // 2 - 1)
      def _(i):
        t0 = 2 * i
        wait(rows0, sem0)
        scat(p, t0, rows0)
        start(p, t0 + 2, rows0, sem0)
        wait(rows1, sem1)
        scat(p, t0 + 1, rows1)
        start(p, t0 + 3, rows1, sem1)

      if g < GROUPS - 1:
        pltpu.make_async_copy(edges.at[0, blk, pl.ds(0, GCH)],
                              src_idx.at[q], semi).wait()
        pltpu.make_async_copy(edges.at[1, blk, pl.ds(0, GCH)],
                              dst_idx.at[q], semi).wait()
        wait(rows0, sem0)
        scat(p, GCH - 2, rows0)
        start(q, 0, rows0, sem0)
        wait(rows1, sem1)
        scat(p, GCH - 1, rows1)
        start(q, 1, rows1, sem1)
      else:
        wait(rows0, sem0)
        scat(p, GCH - 2, rows0)
        wait(rows1, sem1)
        scat(p, GCH - 1, rows1)

    plsc.subcore_barrier()

    pltpu.sync_copy(acc_sh.at[pl.ds(base, ROWS_PER_TILE)],
                    acc_out.at[cid, pl.ds(base, ROWS_PER_TILE)])
    if with_counts:
      pltpu.sync_copy(cnt_sh.at[pl.ds(base, ROWS_PER_TILE)],
                      cnt_out.at[cid, pl.ds(base, ROWS_PER_TILE)])

  out_type = [jax.ShapeDtypeStruct((NC, N, D), jnp.float32)]
  scratch = [
      pltpu.VMEM((2, GCH, B), jnp.int32),
      pltpu.VMEM((2, GCH, B), jnp.int32),
      pltpu.VMEM((B, D), jnp.float32),
      pltpu.VMEM((B, D), jnp.float32),
      pltpu.SemaphoreType.DMA,
      pltpu.SemaphoreType.DMA,
      pltpu.SemaphoreType.DMA,
      pltpu.VMEM_SHARED((N, D), jnp.float32),
  ]
  if with_counts:
    out_type.append(jax.ShapeDtypeStruct((NC, N, 16), jnp.float32))
    scratch.append(pltpu.VMEM((B, 16), jnp.float32))
    scratch.append(pltpu.VMEM_SHARED((N, 16), jnp.float32))

  return pl.kernel(body, out_type=out_type, mesh=_MESH,
                   scratch_types=scratch, compiler_params=_SC_PARAMS)


_sc_agg_cnt = _make_sc_agg(True)
_sc_agg = _make_sc_agg(False)


_RB = 1000
_GRID = N // _RB


def _dot(a, b):
  return lax.dot_general(a, b, (((1,), (0,)), ((), ())),
                         precision=lax.Precision.HIGHEST,
                         preferred_element_type=jnp.float32)


def _mm2_body(x_ref, wl_ref, wr_ref, ol_ref, or_ref):
  xb = x_ref[...]
  ol_ref[...] = _dot(xb, wl_ref[...])
  or_ref[...] = _dot(xb, wr_ref[...])


@jax.jit
def _mm2(x, wl, wr):
  return pl.pallas_call(
      _mm2_body,
      grid=(_GRID,),
      in_specs=[
          pl.BlockSpec((_RB, D), lambda i: (i, 0)),
          pl.BlockSpec((D, D), lambda i: (0, 0)),
          pl.BlockSpec((D, D), lambda i: (0, 0)),
      ],
      out_specs=[
          pl.BlockSpec((_RB, D), lambda i: (i, 0)),
          pl.BlockSpec((_RB, D), lambda i: (i, 0)),
      ],
      out_shape=[
          jax.ShapeDtypeStruct((N, D), jnp.float32),
          jax.ShapeDtypeStruct((N, D), jnp.float32),
      ],
  )(x, wl, wr)


def _mid_body(acc_ref, cnt_ref, xr_ref, b1_ref, wl_ref, wr_ref,
              ol_ref, or_ref):
  s = acc_ref[0] + acc_ref[1]
  c = cnt_ref[0, :, 0] + cnt_ref[1, :, 0]
  rc = 1.0 / jnp.maximum(c, 1.0)
  h = jnp.maximum(s * rc[:, None] + b1_ref[...] + xr_ref[...], 0.0)
  ol_ref[...] = _dot(h, wl_ref[...])
  or_ref[...] = _dot(h, wr_ref[...])


@jax.jit
def _mid(acc, cnt, xr, b1, wl, wr):
  return pl.pallas_call(
      _mid_body,
      grid=(_GRID,),
      in_specs=[
          pl.BlockSpec((NC, _RB, D), lambda i: (0, i, 0)),
          pl.BlockSpec((NC, _RB, 16), lambda i: (0, i, 0)),
          pl.BlockSpec((_RB, D), lambda i: (i, 0)),
          pl.BlockSpec((1, D), lambda i: (0, 0)),
          pl.BlockSpec((D, D), lambda i: (0, 0)),
          pl.BlockSpec((D, D), lambda i: (0, 0)),
      ],
      out_specs=[
          pl.BlockSpec((_RB, D), lambda i: (i, 0)),
          pl.BlockSpec((_RB, D), lambda i: (i, 0)),
      ],
      out_shape=[
          jax.ShapeDtypeStruct((N, D), jnp.float32),
          jax.ShapeDtypeStruct((N, D), jnp.float32),
      ],
  )(acc, cnt, xr, b1, wl, wr)


def _final_body(acc_ref, cnt_ref, hr_ref, b2_ref, o_ref):
  s = acc_ref[0] + acc_ref[1]
  c = cnt_ref[0, :, 0] + cnt_ref[1, :, 0]
  rc = 1.0 / jnp.maximum(c, 1.0)
  o_ref[...] = s * rc[:, None] + b2_ref[...] + hr_ref[...]


@jax.jit
def _final(acc, cnt, hr, b2):
  return pl.pallas_call(
      _final_body,
      grid=(_GRID,),
      in_specs=[
          pl.BlockSpec((NC, _RB, D), lambda i: (0, i, 0)),
          pl.BlockSpec((NC, _RB, 16), lambda i: (0, i, 0)),
          pl.BlockSpec((_RB, D), lambda i: (i, 0)),
          pl.BlockSpec((1, D), lambda i: (0, 0)),
      ],
      out_specs=pl.BlockSpec((_RB, D), lambda i: (i, 0)),
      out_shape=jax.ShapeDtypeStruct((N, D), jnp.float32),
  )(acc, cnt, hr, b2)


@jax.jit
def kernel(x, edge_index, W1_l, b1_l, W1_r, W2_l, b2_l, W2_r):
  edges = edge_index.reshape(2, NW, CHUNKS, B)
  xl, xr = _mm2(x, W1_l, W1_r)
  acc1, cnt = _sc_agg_cnt(xl, edges)
  h2l, h2r = _mid(acc1, cnt, xr, b1_l.reshape(1, D), W2_l, W2_r)
  acc2, = _sc_agg(h2l, edges)
  return _final(acc2, cnt, h2r, b2_l.reshape(1, D))


# speedup vs baseline: 1.0098x; 1.0098x over previous
import functools

import jax
import jax.numpy as jnp
from jax import lax
from jax.experimental import pallas as pl
from jax.experimental.pallas import tpu as pltpu
from jax.experimental.pallas import tpu_sc as plsc

N = 10000
E = 320000
D = 128

NC = 2
NS = 16
NW = NC * NS

B = 100
CHUNKS = 100
ROWS_PER_TILE = N // NS


def _zero_fill(ref, nrows, ncols):
  @pl.loop(0, nrows)
  def _(r):
    for k in range(ncols // 16):
      ref[r, pl.ds(16 * k, 16)] = jnp.zeros((16,), jnp.float32)


_MESH = plsc.VectorSubcoreMesh(core_axis_name="c", subcore_axis_name="s")
_SC_PARAMS = pltpu.CompilerParams(use_tc_tiling_on_sc=False)

GROUPS = 5
GCH = CHUNKS // GROUPS


def _make_sc_agg(with_counts):
  def body(feats, edges, *rest):
    if with_counts:
      (acc_out, cnt_out, src_idx, dst_idx, rows0, rows1,
       sem0, sem1, semi, acc_sh, ones_v, cnt_sh, semc) = rest
    else:
      (acc_out, src_idx, dst_idx, rows0, rows1,
       sem0, sem1, semi, acc_sh) = rest
    cid = lax.axis_index("c")
    sid = lax.axis_index("s")
    blk = cid * NS + sid
    base = sid * ROWS_PER_TILE

    _zero_fill(rows0, B, D)
    nfull = ROWS_PER_TILE // B
    tail = ROWS_PER_TILE - nfull * B
    for k in range(nfull):
      pltpu.sync_copy(rows0, acc_sh.at[pl.ds(base + k * B, B)])
    if tail:
      pltpu.sync_copy(rows0.at[pl.ds(0, tail)],
                      acc_sh.at[pl.ds(base + nfull * B, tail)])
    if with_counts:
      _zero_fill(ones_v, B, 16)
      for k in range(nfull):
        pltpu.sync_copy(ones_v, cnt_sh.at[pl.ds(base + k * B, B)])
      if tail:
        pltpu.sync_copy(ones_v.at[pl.ds(0, tail)],
                        cnt_sh.at[pl.ds(base + nfull * B, tail)])
      @pl.loop(0, B)
      def _(r):
        ones_v[r, :] = jnp.ones((16,), jnp.float32)

    pltpu.sync_copy(edges.at[0, blk, pl.ds(0, GCH)], src_idx.at[0])
    pltpu.sync_copy(edges.at[1, blk, pl.ds(0, GCH)], dst_idx.at[0])

    plsc.subcore_barrier()

    def start(p, t, buf, sem):
      pltpu.async_copy(feats.at[src_idx.at[p, t]], buf, sem)

    def wait(buf, sem):
      pltpu.make_async_copy(feats.at[pl.ds(0, B)], buf, sem).wait()

    def scat(p, t, buf):
      pltpu.sync_copy(buf, acc_sh.at[dst_idx.at[p, t]], add=True)
      if with_counts:
        pltpu.async_copy(ones_v, cnt_sh.at[dst_idx.at[p, t]], semc, add=True)

    start(0, 0, rows0, sem0)
    start(0, 1, rows1, sem1)

    for g in range(GROUPS):
      p = g % 2
      q = 1 - p
      if g < GROUPS - 1:
        pltpu.async_copy(edges.at[0, blk, pl.ds((g + 1) * GCH, GCH)],
                         src_idx.at[q], semi)
        pltpu.async_copy(edges.at[1, blk, pl.ds((g + 1) * GCH, GCH)],
                         dst_idx.at[q], semi)

      @pl.loop(0, GCH // 2 - 1)
      def _(i):
        t0 = 2 * i
        wait(rows0, sem0)
        scat(p, t0, rows0)
        start(p, t0 + 2, rows0, sem0)
        wait(rows1, sem1)
        scat(p, t0 + 1, rows1)
        start(p, t0 + 3, rows1, sem1)

      if g < GROUPS - 1:
        pltpu.make_async_copy(edges.at[0, blk, pl.ds(0, GCH)],
                              src_idx.at[q], semi).wait()
        pltpu.make_async_copy(edges.at[1, blk, pl.ds(0, GCH)],
                              dst_idx.at[q], semi).wait()
        wait(rows0, sem0)
        scat(p, GCH - 2, rows0)
        start(q, 0, rows0, sem0)
        wait(rows1, sem1)
        scat(p, GCH - 1, rows1)
        start(q, 1, rows1, sem1)
      else:
        wait(rows0, sem0)
        scat(p, GCH - 2, rows0)
        wait(rows1, sem1)
        scat(p, GCH - 1, rows1)

    if with_counts:
      @pl.loop(0, CHUNKS)
      def _(j):
        pltpu.make_async_copy(ones_v, cnt_sh.at[pl.ds(0, B)], semc).wait()

    plsc.subcore_barrier()

    pltpu.sync_copy(acc_sh.at[pl.ds(base, ROWS_PER_TILE)],
                    acc_out.at[cid, pl.ds(base, ROWS_PER_TILE)])
    if with_counts:
      pltpu.sync_copy(cnt_sh.at[pl.ds(base, ROWS_PER_TILE)],
                      cnt_out.at[cid, pl.ds(base, ROWS_PER_TILE)])

  out_type = [jax.ShapeDtypeStruct((NC, N, D), jnp.float32)]
  scratch = [
      pltpu.VMEM((2, GCH, B), jnp.int32),
      pltpu.VMEM((2, GCH, B), jnp.int32),
      pltpu.VMEM((B, D), jnp.float32),
      pltpu.VMEM((B, D), jnp.float32),
      pltpu.SemaphoreType.DMA,
      pltpu.SemaphoreType.DMA,
      pltpu.SemaphoreType.DMA,
      pltpu.VMEM_SHARED((N, D), jnp.float32),
  ]
  if with_counts:
    out_type.append(jax.ShapeDtypeStruct((NC, N, 16), jnp.float32))
    scratch.append(pltpu.VMEM((B, 16), jnp.float32))
    scratch.append(pltpu.VMEM_SHARED((N, 16), jnp.float32))
    scratch.append(pltpu.SemaphoreType.DMA)

  return pl.kernel(body, out_type=out_type, mesh=_MESH,
                   scratch_types=scratch, compiler_params=_SC_PARAMS)


_sc_agg_cnt = _make_sc_agg(True)
_sc_agg = _make_sc_agg(False)


_RB = 1000
_GRID = N // _RB


def _dot(a, b):
  return lax.dot_general(a, b, (((1,), (0,)), ((), ())),
                         precision=lax.Precision.HIGHEST,
                         preferred_element_type=jnp.float32)


def _mm2_body(x_ref, wl_ref, wr_ref, ol_ref, or_ref):
  xb = x_ref[...]
  ol_ref[...] = _dot(xb, wl_ref[...])
  or_ref[...] = _dot(xb, wr_ref[...])


@jax.jit
def _mm2(x, wl, wr):
  return pl.pallas_call(
      _mm2_body,
      grid=(_GRID,),
      in_specs=[
          pl.BlockSpec((_RB, D), lambda i: (i, 0)),
          pl.BlockSpec((D, D), lambda i: (0, 0)),
          pl.BlockSpec((D, D), lambda i: (0, 0)),
      ],
      out_specs=[
          pl.BlockSpec((_RB, D), lambda i: (i, 0)),
          pl.BlockSpec((_RB, D), lambda i: (i, 0)),
      ],
      out_shape=[
          jax.ShapeDtypeStruct((N, D), jnp.float32),
          jax.ShapeDtypeStruct((N, D), jnp.float32),
      ],
  )(x, wl, wr)


def _mid_body(acc_ref, cnt_ref, xr_ref, b1_ref, wl_ref, wr_ref,
              ol_ref, or_ref):
  s = acc_ref[0] + acc_ref[1]
  c = cnt_ref[0, :, 0] + cnt_ref[1, :, 0]
  rc = 1.0 / jnp.maximum(c, 1.0)
  h = jnp.maximum(s * rc[:, None] + b1_ref[...] + xr_ref[...], 0.0)
  ol_ref[...] = _dot(h, wl_ref[...])
  or_ref[...] = _dot(h, wr_ref[...])


@jax.jit
def _mid(acc, cnt, xr, b1, wl, wr):
  return pl.pallas_call(
      _mid_body,
      grid=(_GRID,),
      in_specs=[
          pl.BlockSpec((NC, _RB, D), lambda i: (0, i, 0)),
          pl.BlockSpec((NC, _RB, 16), lambda i: (0, i, 0)),
          pl.BlockSpec((_RB, D), lambda i: (i, 0)),
          pl.BlockSpec((1, D), lambda i: (0, 0)),
          pl.BlockSpec((D, D), lambda i: (0, 0)),
          pl.BlockSpec((D, D), lambda i: (0, 0)),
      ],
      out_specs=[
          pl.BlockSpec((_RB, D), lambda i: (i, 0)),
          pl.BlockSpec((_RB, D), lambda i: (i, 0)),
      ],
      out_shape=[
          jax.ShapeDtypeStruct((N, D), jnp.float32),
          jax.ShapeDtypeStruct((N, D), jnp.float32),
      ],
  )(acc, cnt, xr, b1, wl, wr)


def _final_body(acc_ref, cnt_ref, hr_ref, b2_ref, o_ref):
  s = acc_ref[0] + acc_ref[1]
  c = cnt_ref[0, :, 0] + cnt_ref[1, :, 0]
  rc = 1.0 / jnp.maximum(c, 1.0)
  o_ref[...] = s * rc[:, None] + b2_ref[...] + hr_ref[...]


@jax.jit
def _final(acc, cnt, hr, b2):
  return pl.pallas_call(
      _final_body,
      grid=(_GRID,),
      in_specs=[
          pl.BlockSpec((NC, _RB, D), lambda i: (0, i, 0)),
          pl.BlockSpec((NC, _RB, 16), lambda i: (0, i, 0)),
          pl.BlockSpec((_RB, D), lambda i: (i, 0)),
          pl.BlockSpec((1, D), lambda i: (0, 0)),
      ],
      out_specs=pl.BlockSpec((_RB, D), lambda i: (i, 0)),
      out_shape=jax.ShapeDtypeStruct((N, D), jnp.float32),
  )(acc, cnt, hr, b2)


@jax.jit
def kernel(x, edge_index, W1_l, b1_l, W1_r, W2_l, b2_l, W2_r):
  edges = edge_index.reshape(2, NW, CHUNKS, B)
  xl, xr = _mm2(x, W1_l, W1_r)
  acc1, cnt = _sc_agg_cnt(xl, edges)
  h2l, h2r = _mid(acc1, cnt, xr, b1_l.reshape(1, D), W2_l, W2_r)
  acc2, = _sc_agg(h2l, edges)
  return _final(acc2, cnt, h2r, b2_l.reshape(1, D))


# speedup vs baseline: 1.0427x; 1.0326x over previous
import functools

import jax
import jax.numpy as jnp
from jax import lax
from jax.experimental import pallas as pl
from jax.experimental.pallas import tpu as pltpu
from jax.experimental.pallas import tpu_sc as plsc

N = 10000
E = 320000
D = 128

NC = 2
NS = 16
NW = NC * NS
EPT = E // NW
ROWS_PER_TILE = N // NS

B = 125
CHUNKS = EPT // B
GROUPS = 5
GCH = CHUNKS // GROUPS

CB = 100
CCH = EPT // CB


def _zero_fill(ref, nrows, ncols):
  @pl.loop(0, nrows)
  def _(r):
    for k in range(ncols // 16):
      ref[r, pl.ds(16 * k, 16)] = jnp.zeros((16,), jnp.float32)


_MESH = plsc.VectorSubcoreMesh(core_axis_name="c", subcore_axis_name="s")
_SC_PARAMS = pltpu.CompilerParams(use_tc_tiling_on_sc=False)


def _sc_agg_body(feats, edges, acc_out,
                 src_idx, dst_idx, rows0, rows1, sem0, sem1, semi, acc_sh):
  cid = lax.axis_index("c")
  sid = lax.axis_index("s")
  blk = cid * NS + sid
  base = sid * ROWS_PER_TILE

  _zero_fill(rows0, B, D)
  nfull = ROWS_PER_TILE // B
  tail = ROWS_PER_TILE - nfull * B
  for k in range(nfull):
    pltpu.sync_copy(rows0, acc_sh.at[pl.ds(base + k * B, B)])
  if tail:
    pltpu.sync_copy(rows0.at[pl.ds(0, tail)],
                    acc_sh.at[pl.ds(base + nfull * B, tail)])

  pltpu.sync_copy(edges.at[0, blk, pl.ds(0, GCH)], src_idx.at[0])
  pltpu.sync_copy(edges.at[1, blk, pl.ds(0, GCH)], dst_idx.at[0])

  plsc.subcore_barrier()

  def start(p, t, buf, sem):
    pltpu.async_copy(feats.at[src_idx.at[p, t]], buf, sem)

  def wait(buf, sem):
    pltpu.make_async_copy(feats.at[pl.ds(0, B)], buf, sem).wait()

  def scat(p, t, buf):
    pltpu.sync_copy(buf, acc_sh.at[dst_idx.at[p, t]], add=True)

  start(0, 0, rows0, sem0)
  start(0, 1, rows1, sem1)

  for g in range(GROUPS):
    p = g % 2
    q = 1 - p
    if g < GROUPS - 1:
      pltpu.async_copy(edges.at[0, blk, pl.ds((g + 1) * GCH, GCH)],
                       src_idx.at[q], semi)
      pltpu.async_copy(edges.at[1, blk, pl.ds((g + 1) * GCH, GCH)],
                       dst_idx.at[q], semi)

    @pl.loop(0, GCH // 2 - 1)
    def _(i):
      t0 = 2 * i
      wait(rows0, sem0)
      scat(p, t0, rows0)
      start(p, t0 + 2, rows0, sem0)
      wait(rows1, sem1)
      scat(p, t0 + 1, rows1)
      start(p, t0 + 3, rows1, sem1)

    if g < GROUPS - 1:
      pltpu.make_async_copy(edges.at[0, blk, pl.ds(0, GCH)],
                            src_idx.at[q], semi).wait()
      pltpu.make_async_copy(edges.at[1, blk, pl.ds(0, GCH)],
                            dst_idx.at[q], semi).wait()
      wait(rows0, sem0)
      scat(p, GCH - 2, rows0)
      start(q, 0, rows0, sem0)
      wait(rows1, sem1)
      scat(p, GCH - 1, rows1)
      start(q, 1, rows1, sem1)
    else:
      wait(rows0, sem0)
      scat(p, GCH - 2, rows0)
      wait(rows1, sem1)
      scat(p, GCH - 1, rows1)

  plsc.subcore_barrier()

  pltpu.sync_copy(acc_sh.at[pl.ds(base, ROWS_PER_TILE)],
                  acc_out.at[cid, pl.ds(base, ROWS_PER_TILE)])


_sc_agg = pl.kernel(
    _sc_agg_body,
    out_type=[jax.ShapeDtypeStruct((NC, N, D), jnp.float32)],
    mesh=_MESH,
    scratch_types=[
        pltpu.VMEM((2, GCH, B), jnp.int32),
        pltpu.VMEM((2, GCH, B), jnp.int32),
        pltpu.VMEM((B, D), jnp.float32),
        pltpu.VMEM((B, D), jnp.float32),
        pltpu.SemaphoreType.DMA,
        pltpu.SemaphoreType.DMA,
        pltpu.SemaphoreType.DMA,
        pltpu.VMEM_SHARED((N, D), jnp.float32),
    ],
    compiler_params=_SC_PARAMS)


def _sc_cnt_body(edges, cnt_out, dst_idx, ones_v, cnt_sh):
  cid = lax.axis_index("c")
  sid = lax.axis_index("s")
  blk = cid * NS + sid
  base = sid * ROWS_PER_TILE

  _zero_fill(ones_v, CB, 16)
  nfull = ROWS_PER_TILE // CB
  tail = ROWS_PER_TILE - nfull * CB
  for k in range(nfull):
    pltpu.sync_copy(ones_v, cnt_sh.at[pl.ds(base + k * CB, CB)])
  if tail:
    pltpu.sync_copy(ones_v.at[pl.ds(0, tail)],
                    cnt_sh.at[pl.ds(base + nfull * CB, tail)])

  @pl.loop(0, CB)
  def _(r):
    ones_v[r, :] = jnp.ones((16,), jnp.float32)

  pltpu.sync_copy(edges.at[1, blk], dst_idx)

  plsc.subcore_barrier()

  @pl.loop(0, CCH)
  def _(j):
    pltpu.sync_copy(ones_v, cnt_sh.at[dst_idx.at[j]], add=True)

  plsc.subcore_barrier()

  pltpu.sync_copy(cnt_sh.at[pl.ds(base, ROWS_PER_TILE)],
                  cnt_out.at[cid, pl.ds(base, ROWS_PER_TILE)])


_sc_cnt = pl.kernel(
    _sc_cnt_body,
    out_type=[jax.ShapeDtypeStruct((NC, N, 16), jnp.float32)],
    mesh=_MESH,
    scratch_types=[
        pltpu.VMEM((CCH, CB), jnp.int32),
        pltpu.VMEM((CB, 16), jnp.float32),
        pltpu.VMEM_SHARED((N, 16), jnp.float32),
    ],
    compiler_params=_SC_PARAMS)


_RB = 1000
_GRID = N // _RB


def _dot(a, b):
  return lax.dot_general(a, b, (((1,), (0,)), ((), ())),
                         precision=lax.Precision.HIGHEST,
                         preferred_element_type=jnp.float32)


def _mm2_body(x_ref, wl_ref, wr_ref, ol_ref, or_ref):
  xb = x_ref[...]
  ol_ref[...] = _dot(xb, wl_ref[...])
  or_ref[...] = _dot(xb, wr_ref[...])


@jax.jit
def _mm2(x, wl, wr):
  return pl.pallas_call(
      _mm2_body,
      grid=(_GRID,),
      in_specs=[
          pl.BlockSpec((_RB, D), lambda i: (i, 0)),
          pl.BlockSpec((D, D), lambda i: (0, 0)),
          pl.BlockSpec((D, D), lambda i: (0, 0)),
      ],
      out_specs=[
          pl.BlockSpec((_RB, D), lambda i: (i, 0)),
          pl.BlockSpec((_RB, D), lambda i: (i, 0)),
      ],
      out_shape=[
          jax.ShapeDtypeStruct((N, D), jnp.float32),
          jax.ShapeDtypeStruct((N, D), jnp.float32),
      ],
  )(x, wl, wr)


def _mid_body(acc_ref, cnt_ref, xr_ref, b1_ref, wl_ref, wr_ref,
              ol_ref, or_ref):
  s = acc_ref[0] + acc_ref[1]
  c = cnt_ref[0, :, 0] + cnt_ref[1, :, 0]
  rc = 1.0 / jnp.maximum(c, 1.0)
  h = jnp.maximum(s * rc[:, None] + b1_ref[...] + xr_ref[...], 0.0)
  ol_ref[...] = _dot(h, wl_ref[...])
  or_ref[...] = _dot(h, wr_ref[...])


@jax.jit
def _mid(acc, cnt, xr, b1, wl, wr):
  return pl.pallas_call(
      _mid_body,
      grid=(_GRID,),
      in_specs=[
          pl.BlockSpec((NC, _RB, D), lambda i: (0, i, 0)),
          pl.BlockSpec((NC, _RB, 16), lambda i: (0, i, 0)),
          pl.BlockSpec((_RB, D), lambda i: (i, 0)),
          pl.BlockSpec((1, D), lambda i: (0, 0)),
          pl.BlockSpec((D, D), lambda i: (0, 0)),
          pl.BlockSpec((D, D), lambda i: (0, 0)),
      ],
      out_specs=[
          pl.BlockSpec((_RB, D), lambda i: (i, 0)),
          pl.BlockSpec((_RB, D), lambda i: (i, 0)),
      ],
      out_shape=[
          jax.ShapeDtypeStruct((N, D), jnp.float32),
          jax.ShapeDtypeStruct((N, D), jnp.float32),
      ],
  )(acc, cnt, xr, b1, wl, wr)


def _final_body(acc_ref, cnt_ref, hr_ref, b2_ref, o_ref):
  s = acc_ref[0] + acc_ref[1]
  c = cnt_ref[0, :, 0] + cnt_ref[1, :, 0]
  rc = 1.0 / jnp.maximum(c, 1.0)
  o_ref[...] = s * rc[:, None] + b2_ref[...] + hr_ref[...]


@jax.jit
def _final(acc, cnt, hr, b2):
  return pl.pallas_call(
      _final_body,
      grid=(_GRID,),
      in_specs=[
          pl.BlockSpec((NC, _RB, D), lambda i: (0, i, 0)),
          pl.BlockSpec((NC, _RB, 16), lambda i: (0, i, 0)),
          pl.BlockSpec((_RB, D), lambda i: (i, 0)),
          pl.BlockSpec((1, D), lambda i: (0, 0)),
      ],
      out_specs=pl.BlockSpec((_RB, D), lambda i: (i, 0)),
      out_shape=jax.ShapeDtypeStruct((N, D), jnp.float32),
  )(acc, cnt, hr, b2)


@jax.jit
def kernel(x, edge_index, W1_l, b1_l, W1_r, W2_l, b2_l, W2_r):
  edges_a = edge_index.reshape(2, NW, CHUNKS, B)
  edges_c = edge_index.reshape(2, NW, CCH, CB)
  xl, xr = _mm2(x, W1_l, W1_r)
  cnt, = _sc_cnt(edges_c)
  acc1, = _sc_agg(xl, edges_a)
  h2l, h2r = _mid(acc1, cnt, xr, b1_l.reshape(1, D), W2_l, W2_r)
  acc2, = _sc_agg(h2l, edges_a)
  return _final(acc2, cnt, h2r, b2_l.reshape(1, D))


# speedup vs baseline: 1.1160x; 1.0702x over previous
import functools

import jax
import jax.numpy as jnp
from jax import lax
from jax.experimental import pallas as pl
from jax.experimental.pallas import tpu as pltpu
from jax.experimental.pallas import tpu_sc as plsc

N = 10000
E = 320000
D = 128

NC = 2
NS = 16
NW = NC * NS
EPT = E // NW
ROWS_PER_TILE = N // NS

B = 125
CHUNKS = EPT // B
GROUPS = 5
GCH = CHUNKS // GROUPS

CB = B
CCH = CHUNKS


def _zero_fill(ref, nrows, ncols):
  @pl.loop(0, nrows)
  def _(r):
    for k in range(ncols // 16):
      ref[r, pl.ds(16 * k, 16)] = jnp.zeros((16,), jnp.float32)


_MESH = plsc.VectorSubcoreMesh(core_axis_name="c", subcore_axis_name="s")
_SC_PARAMS = pltpu.CompilerParams(use_tc_tiling_on_sc=False)


def _sc_agg_body(feats, edges, acc_out,
                 src_idx, dst_idx, rows0, rows1, sem0, sem1, semi, acc_sh):
  cid = lax.axis_index("c")
  sid = lax.axis_index("s")
  blk = cid * NS + sid
  base = sid * ROWS_PER_TILE

  pltpu.sync_copy(edges.at[0, blk, pl.ds(0, GCH)], src_idx.at[0])
  pltpu.sync_copy(edges.at[1, blk, pl.ds(0, GCH)], dst_idx.at[0])

  _zero_fill(rows0, B, D)
  nfull = ROWS_PER_TILE // B
  tail = ROWS_PER_TILE - nfull * B
  for k in range(nfull):
    pltpu.async_copy(rows0, acc_sh.at[pl.ds(base + k * B, B)], semi)
  if tail:
    pltpu.async_copy(rows0.at[pl.ds(0, tail)],
                     acc_sh.at[pl.ds(base + nfull * B, tail)], semi)
  for k in range(nfull):
    pltpu.make_async_copy(rows0, acc_sh.at[pl.ds(base, B)], semi).wait()
  if tail:
    pltpu.make_async_copy(rows0.at[pl.ds(0, tail)],
                          acc_sh.at[pl.ds(base, tail)], semi).wait()

  plsc.subcore_barrier()

  def start(p, t, buf, sem):
    pltpu.async_copy(feats.at[src_idx.at[p, t]], buf, sem)

  def wait(buf, sem):
    pltpu.make_async_copy(feats.at[pl.ds(0, B)], buf, sem).wait()

  def scat(p, t, buf):
    pltpu.sync_copy(buf, acc_sh.at[dst_idx.at[p, t]], add=True)

  start(0, 0, rows0, sem0)
  start(0, 1, rows1, sem1)

  for g in range(GROUPS):
    p = g % 2
    q = 1 - p
    if g < GROUPS - 1:
      pltpu.async_copy(edges.at[0, blk, pl.ds((g + 1) * GCH, GCH)],
                       src_idx.at[q], semi)
      pltpu.async_copy(edges.at[1, blk, pl.ds((g + 1) * GCH, GCH)],
                       dst_idx.at[q], semi)

    @pl.loop(0, GCH // 2 - 1)
    def _(i):
      t0 = 2 * i
      wait(rows0, sem0)
      scat(p, t0, rows0)
      start(p, t0 + 2, rows0, sem0)
      wait(rows1, sem1)
      scat(p, t0 + 1, rows1)
      start(p, t0 + 3, rows1, sem1)

    if g < GROUPS - 1:
      pltpu.make_async_copy(edges.at[0, blk, pl.ds(0, GCH)],
                            src_idx.at[q], semi).wait()
      pltpu.make_async_copy(edges.at[1, blk, pl.ds(0, GCH)],
                            dst_idx.at[q], semi).wait()
      wait(rows0, sem0)
      scat(p, GCH - 2, rows0)
      start(q, 0, rows0, sem0)
      wait(rows1, sem1)
      scat(p, GCH - 1, rows1)
      start(q, 1, rows1, sem1)
    else:
      wait(rows0, sem0)
      scat(p, GCH - 2, rows0)
      wait(rows1, sem1)
      scat(p, GCH - 1, rows1)

  plsc.subcore_barrier()

  pltpu.sync_copy(acc_sh.at[pl.ds(base, ROWS_PER_TILE)],
                  acc_out.at[cid, pl.ds(base, ROWS_PER_TILE)])


_sc_agg = pl.kernel(
    _sc_agg_body,
    out_type=[jax.ShapeDtypeStruct((NC, N, D), jnp.float32)],
    mesh=_MESH,
    scratch_types=[
        pltpu.VMEM((2, GCH, B), jnp.int32),
        pltpu.VMEM((2, GCH, B), jnp.int32),
        pltpu.VMEM((B, D), jnp.float32),
        pltpu.VMEM((B, D), jnp.float32),
        pltpu.SemaphoreType.DMA,
        pltpu.SemaphoreType.DMA,
        pltpu.SemaphoreType.DMA,
        pltpu.VMEM_SHARED((N, D), jnp.float32),
    ],
    compiler_params=_SC_PARAMS)


def _sc_cnt_body(edges, cnt_out, dst_idx, ones_v, cnt_sh):
  cid = lax.axis_index("c")
  sid = lax.axis_index("s")
  blk = cid * NS + sid
  base = sid * ROWS_PER_TILE

  _zero_fill(ones_v, CB, 16)
  nfull = ROWS_PER_TILE // CB
  tail = ROWS_PER_TILE - nfull * CB
  for k in range(nfull):
    pltpu.sync_copy(ones_v, cnt_sh.at[pl.ds(base + k * CB, CB)])
  if tail:
    pltpu.sync_copy(ones_v.at[pl.ds(0, tail)],
                    cnt_sh.at[pl.ds(base + nfull * CB, tail)])

  @pl.loop(0, CB)
  def _(r):
    ones_v[r, :] = jnp.ones((16,), jnp.float32)

  pltpu.sync_copy(edges.at[1, blk], dst_idx)

  plsc.subcore_barrier()

  @pl.loop(0, CCH)
  def _(j):
    pltpu.sync_copy(ones_v, cnt_sh.at[dst_idx.at[j]], add=True)

  plsc.subcore_barrier()

  pltpu.sync_copy(cnt_sh.at[pl.ds(base, ROWS_PER_TILE)],
                  cnt_out.at[cid, pl.ds(base, ROWS_PER_TILE)])


_sc_cnt = pl.kernel(
    _sc_cnt_body,
    out_type=[jax.ShapeDtypeStruct((NC, N, 16), jnp.float32)],
    mesh=_MESH,
    scratch_types=[
        pltpu.VMEM((CCH, CB), jnp.int32),
        pltpu.VMEM((CB, 16), jnp.float32),
        pltpu.VMEM_SHARED((N, 16), jnp.float32),
    ],
    compiler_params=_SC_PARAMS)


_RB = 2000
_GRID = N // _RB


def _dot(a, b):
  return lax.dot_general(a, b, (((1,), (0,)), ((), ())),
                         precision=lax.Precision.HIGHEST,
                         preferred_element_type=jnp.float32)


def _mm_body(x_ref, w_ref, o_ref):
  o_ref[...] = _dot(x_ref[...], w_ref[...])


@jax.jit
def _mm(x, w):
  return pl.pallas_call(
      _mm_body,
      grid=(_GRID,),
      in_specs=[
          pl.BlockSpec((_RB, D), lambda i: (i, 0)),
          pl.BlockSpec((D, D), lambda i: (0, 0)),
      ],
      out_specs=pl.BlockSpec((_RB, D), lambda i: (i, 0)),
      out_shape=jax.ShapeDtypeStruct((N, D), jnp.float32),
  )(x, w)


def _mid_body(acc_ref, cnt_ref, xr_ref, b1_ref, wl_ref, wr_ref,
              ol_ref, or_ref):
  s = acc_ref[0] + acc_ref[1]
  c = cnt_ref[0, :, 0] + cnt_ref[1, :, 0]
  rc = 1.0 / jnp.maximum(c, 1.0)
  h = jnp.maximum(s * rc[:, None] + b1_ref[...] + xr_ref[...], 0.0)
  ol_ref[...] = _dot(h, wl_ref[...])
  or_ref[...] = _dot(h, wr_ref[...])


@jax.jit
def _mid(acc, cnt, xr, b1, wl, wr):
  return pl.pallas_call(
      _mid_body,
      grid=(_GRID,),
      in_specs=[
          pl.BlockSpec((NC, _RB, D), lambda i: (0, i, 0)),
          pl.BlockSpec((NC, _RB, 16), lambda i: (0, i, 0)),
          pl.BlockSpec((_RB, D), lambda i: (i, 0)),
          pl.BlockSpec((1, D), lambda i: (0, 0)),
          pl.BlockSpec((D, D), lambda i: (0, 0)),
          pl.BlockSpec((D, D), lambda i: (0, 0)),
      ],
      out_specs=[
          pl.BlockSpec((_RB, D), lambda i: (i, 0)),
          pl.BlockSpec((_RB, D), lambda i: (i, 0)),
      ],
      out_shape=[
          jax.ShapeDtypeStruct((N, D), jnp.float32),
          jax.ShapeDtypeStruct((N, D), jnp.float32),
      ],
  )(acc, cnt, xr, b1, wl, wr)


def _final_body(acc_ref, cnt_ref, hr_ref, b2_ref, o_ref):
  s = acc_ref[0] + acc_ref[1]
  c = cnt_ref[0, :, 0] + cnt_ref[1, :, 0]
  rc = 1.0 / jnp.maximum(c, 1.0)
  o_ref[...] = s * rc[:, None] + b2_ref[...] + hr_ref[...]


@jax.jit
def _final(acc, cnt, hr, b2):
  return pl.pallas_call(
      _final_body,
      grid=(_GRID,),
      in_specs=[
          pl.BlockSpec((NC, _RB, D), lambda i: (0, i, 0)),
          pl.BlockSpec((NC, _RB, 16), lambda i: (0, i, 0)),
          pl.BlockSpec((_RB, D), lambda i: (i, 0)),
          pl.BlockSpec((1, D), lambda i: (0, 0)),
      ],
      out_specs=pl.BlockSpec((_RB, D), lambda i: (i, 0)),
      out_shape=jax.ShapeDtypeStruct((N, D), jnp.float32),
  )(acc, cnt, hr, b2)


@jax.jit
def kernel(x, edge_index, W1_l, b1_l, W1_r, W2_l, b2_l, W2_r):
  edges_a = edge_index.reshape(2, NW, CHUNKS, B)
  xl = _mm(x, W1_l)
  xr = _mm(x, W1_r)
  cnt, = _sc_cnt(edges_a)
  acc1, = _sc_agg(xl, edges_a)
  h2l, h2r = _mid(acc1, cnt, xr, b1_l.reshape(1, D), W2_l, W2_r)
  acc2, = _sc_agg(h2l, edges_a)
  return _final(acc2, cnt, h2r, b2_l.reshape(1, D))


# speedup vs baseline: 1.1304x; 1.0129x over previous
import functools

import jax
import jax.numpy as jnp
from jax import lax
from jax.experimental import pallas as pl
from jax.experimental.pallas import tpu as pltpu
from jax.experimental.pallas import tpu_sc as plsc

N = 10000
E = 320000
D = 128

NC = 2
NS = 16
NW = NC * NS
EPT = E // NW
ROWS_PER_TILE = N // NS

B = 125
CHUNKS = EPT // B
GROUPS = 5
GCH = CHUNKS // GROUPS

CB = B
CCH = CHUNKS


def _zero_fill(ref, nrows, ncols):
  @pl.loop(0, nrows)
  def _(r):
    for k in range(ncols // 16):
      ref[r, pl.ds(16 * k, 16)] = jnp.zeros((16,), jnp.float32)


_MESH = plsc.VectorSubcoreMesh(core_axis_name="c", subcore_axis_name="s")
_SC_PARAMS = pltpu.CompilerParams(use_tc_tiling_on_sc=False)


def _sc_agg_body(feats, edges, acc_out,
                 src_idx, dst_idx, rows0, rows1, sem0, sem1, semi, acc_sh):
  cid = lax.axis_index("c")
  sid = lax.axis_index("s")
  blk = cid * NS + sid
  base = sid * ROWS_PER_TILE

  pltpu.sync_copy(edges.at[0, blk, pl.ds(0, GCH)], src_idx.at[0])
  pltpu.sync_copy(edges.at[1, blk, pl.ds(0, GCH)], dst_idx.at[0])

  _zero_fill(rows0, B, D)
  nfull = ROWS_PER_TILE // B
  tail = ROWS_PER_TILE - nfull * B
  for k in range(nfull):
    pltpu.async_copy(rows0, acc_sh.at[pl.ds(base + k * B, B)], semi)
  if tail:
    pltpu.async_copy(rows0.at[pl.ds(0, tail)],
                     acc_sh.at[pl.ds(base + nfull * B, tail)], semi)
  for k in range(nfull):
    pltpu.make_async_copy(rows0, acc_sh.at[pl.ds(base, B)], semi).wait()
  if tail:
    pltpu.make_async_copy(rows0.at[pl.ds(0, tail)],
                          acc_sh.at[pl.ds(base, tail)], semi).wait()

  plsc.subcore_barrier()

  def start(p, t, buf, sem):
    pltpu.async_copy(feats.at[src_idx.at[p, t]], buf, sem)

  def wait(buf, sem):
    pltpu.make_async_copy(feats.at[pl.ds(0, B)], buf, sem).wait()

  def scat(p, t, buf):
    pltpu.sync_copy(buf, acc_sh.at[dst_idx.at[p, t]], add=True)

  start(0, 0, rows0, sem0)
  start(0, 1, rows1, sem1)

  for g in range(GROUPS):
    p = g % 2
    q = 1 - p
    if g < GROUPS - 1:
      pltpu.async_copy(edges.at[0, blk, pl.ds((g + 1) * GCH, GCH)],
                       src_idx.at[q], semi)
      pltpu.async_copy(edges.at[1, blk, pl.ds((g + 1) * GCH, GCH)],
                       dst_idx.at[q], semi)

    @pl.loop(0, GCH // 2 - 1)
    def _(i):
      t0 = 2 * i
      wait(rows0, sem0)
      scat(p, t0, rows0)
      start(p, t0 + 2, rows0, sem0)
      wait(rows1, sem1)
      scat(p, t0 + 1, rows1)
      start(p, t0 + 3, rows1, sem1)

    if g < GROUPS - 1:
      pltpu.make_async_copy(edges.at[0, blk, pl.ds(0, GCH)],
                            src_idx.at[q], semi).wait()
      pltpu.make_async_copy(edges.at[1, blk, pl.ds(0, GCH)],
                            dst_idx.at[q], semi).wait()
      wait(rows0, sem0)
      scat(p, GCH - 2, rows0)
      start(q, 0, rows0, sem0)
      wait(rows1, sem1)
      scat(p, GCH - 1, rows1)
      start(q, 1, rows1, sem1)
    else:
      wait(rows0, sem0)
      scat(p, GCH - 2, rows0)
      wait(rows1, sem1)
      scat(p, GCH - 1, rows1)

  plsc.subcore_barrier()

  pltpu.sync_copy(acc_sh.at[pl.ds(base, ROWS_PER_TILE)],
                  acc_out.at[cid, pl.ds(base, ROWS_PER_TILE)])


_sc_agg = pl.kernel(
    _sc_agg_body,
    out_type=[jax.ShapeDtypeStruct((NC, N, D), jnp.float32)],
    mesh=_MESH,
    scratch_types=[
        pltpu.VMEM((2, GCH, B), jnp.int32),
        pltpu.VMEM((2, GCH, B), jnp.int32),
        pltpu.VMEM((B, D), jnp.float32),
        pltpu.VMEM((B, D), jnp.float32),
        pltpu.SemaphoreType.DMA,
        pltpu.SemaphoreType.DMA,
        pltpu.SemaphoreType.DMA,
        pltpu.VMEM_SHARED((N, D), jnp.float32),
    ],
    compiler_params=_SC_PARAMS)


def _sc_cnt_body(edges, cnt_out, dst_idx, ones_v, cnt_sh):
  cid = lax.axis_index("c")
  sid = lax.axis_index("s")
  blk = cid * NS + sid
  base = sid * ROWS_PER_TILE

  _zero_fill(ones_v, CB, 16)
  nfull = ROWS_PER_TILE // CB
  tail = ROWS_PER_TILE - nfull * CB
  for k in range(nfull):
    pltpu.sync_copy(ones_v, cnt_sh.at[pl.ds(base + k * CB, CB)])
  if tail:
    pltpu.sync_copy(ones_v.at[pl.ds(0, tail)],
                    cnt_sh.at[pl.ds(base + nfull * CB, tail)])

  @pl.loop(0, CB)
  def _(r):
    ones_v[r, :] = jnp.ones((16,), jnp.float32)

  pltpu.sync_copy(edges.at[1, blk], dst_idx)

  plsc.subcore_barrier()

  @pl.loop(0, CCH)
  def _(j):
    pltpu.sync_copy(ones_v, cnt_sh.at[dst_idx.at[j]], add=True)

  plsc.subcore_barrier()

  pltpu.sync_copy(cnt_sh.at[pl.ds(base, ROWS_PER_TILE)],
                  cnt_out.at[cid, pl.ds(base, ROWS_PER_TILE)])


_sc_cnt = pl.kernel(
    _sc_cnt_body,
    out_type=[jax.ShapeDtypeStruct((NC, N, 16), jnp.float32)],
    mesh=_MESH,
    scratch_types=[
        pltpu.VMEM((CCH, CB), jnp.int32),
        pltpu.VMEM((CB, 16), jnp.float32),
        pltpu.VMEM_SHARED((N, 16), jnp.float32),
    ],
    compiler_params=_SC_PARAMS)


_RB = 2000
_GRID = N // _RB


def _dot(a, b):
  return lax.dot_general(a, b, (((1,), (0,)), ((), ())),
                         precision=lax.Precision.DEFAULT,
                         preferred_element_type=jnp.float32)


def _mm_body(x_ref, w_ref, o_ref):
  o_ref[...] = _dot(x_ref[...], w_ref[...])


@jax.jit
def _mm(x, w):
  return pl.pallas_call(
      _mm_body,
      grid=(_GRID,),
      in_specs=[
          pl.BlockSpec((_RB, D), lambda i: (i, 0)),
          pl.BlockSpec((D, D), lambda i: (0, 0)),
      ],
      out_specs=pl.BlockSpec((_RB, D), lambda i: (i, 0)),
      out_shape=jax.ShapeDtypeStruct((N, D), jnp.float32),
  )(x, w)


def _mid_body(acc_ref, cnt_ref, xr_ref, b1_ref, wl_ref, wr_ref,
              ol_ref, or_ref):
  s = acc_ref[0] + acc_ref[1]
  c = cnt_ref[0, :, 0] + cnt_ref[1, :, 0]
  rc = 1.0 / jnp.maximum(c, 1.0)
  h = jnp.maximum(s * rc[:, None] + b1_ref[...] + xr_ref[...], 0.0)
  ol_ref[...] = _dot(h, wl_ref[...])
  or_ref[...] = _dot(h, wr_ref[...])


@jax.jit
def _mid(acc, cnt, xr, b1, wl, wr):
  return pl.pallas_call(
      _mid_body,
      grid=(_GRID,),
      in_specs=[
          pl.BlockSpec((NC, _RB, D), lambda i: (0, i, 0)),
          pl.BlockSpec((NC, _RB, 16), lambda i: (0, i, 0)),
          pl.BlockSpec((_RB, D), lambda i: (i, 0)),
          pl.BlockSpec((1, D), lambda i: (0, 0)),
          pl.BlockSpec((D, D), lambda i: (0, 0)),
          pl.BlockSpec((D, D), lambda i: (0, 0)),
      ],
      out_specs=[
          pl.BlockSpec((_RB, D), lambda i: (i, 0)),
          pl.BlockSpec((_RB, D), lambda i: (i, 0)),
      ],
      out_shape=[
          jax.ShapeDtypeStruct((N, D), jnp.float32),
          jax.ShapeDtypeStruct((N, D), jnp.float32),
      ],
  )(acc, cnt, xr, b1, wl, wr)


def _final_body(acc_ref, cnt_ref, hr_ref, b2_ref, o_ref):
  s = acc_ref[0] + acc_ref[1]
  c = cnt_ref[0, :, 0] + cnt_ref[1, :, 0]
  rc = 1.0 / jnp.maximum(c, 1.0)
  o_ref[...] = s * rc[:, None] + b2_ref[...] + hr_ref[...]


@jax.jit
def _final(acc, cnt, hr, b2):
  return pl.pallas_call(
      _final_body,
      grid=(_GRID,),
      in_specs=[
          pl.BlockSpec((NC, _RB, D), lambda i: (0, i, 0)),
          pl.BlockSpec((NC, _RB, 16), lambda i: (0, i, 0)),
          pl.BlockSpec((_RB, D), lambda i: (i, 0)),
          pl.BlockSpec((1, D), lambda i: (0, 0)),
      ],
      out_specs=pl.BlockSpec((_RB, D), lambda i: (i, 0)),
      out_shape=jax.ShapeDtypeStruct((N, D), jnp.float32),
  )(acc, cnt, hr, b2)


@jax.jit
def kernel(x, edge_index, W1_l, b1_l, W1_r, W2_l, b2_l, W2_r):
  edges_a = edge_index.reshape(2, NW, CHUNKS, B)
  xl = _mm(x, W1_l)
  xr = _mm(x, W1_r)
  cnt, = _sc_cnt(edges_a)
  acc1, = _sc_agg(xl, edges_a)
  h2l, h2r = _mid(acc1, cnt, xr, b1_l.reshape(1, D), W2_l, W2_r)
  acc2, = _sc_agg(h2l, edges_a)
  return _final(acc2, cnt, h2r, b2_l.reshape(1, D))


# speedup vs baseline: 1.1601x; 1.0263x over previous
import functools

import jax
import jax.numpy as jnp
from jax import lax
from jax.experimental import pallas as pl
from jax.experimental.pallas import tpu as pltpu
from jax.experimental.pallas import tpu_sc as plsc

N = 10000
E = 320000
D = 128

NC = 2
NS = 16
NW = NC * NS
EPT = E // NW
ROWS_PER_TILE = N // NS

B = 125
CHUNKS = EPT // B
GROUPS = 5
GCH = CHUNKS // GROUPS


def _zero_fill(ref, nrows, ncols):
  @pl.loop(0, nrows)
  def _(r):
    for k in range(ncols // 16):
      ref[r, pl.ds(16 * k, 16)] = jnp.zeros((16,), jnp.float32)


_MESH = plsc.VectorSubcoreMesh(core_axis_name="c", subcore_axis_name="s")
_SC_PARAMS = pltpu.CompilerParams(use_tc_tiling_on_sc=False,
                                  needs_layout_passes=False)


def _sc_agg_body(feats, edges, acc_out,
                 src_idx, dst_idx, rows0, rows1, sem0, sem1, semi, acc_sh):
  cid = lax.axis_index("c")
  sid = lax.axis_index("s")
  blk = cid * NS + sid
  base = sid * ROWS_PER_TILE

  pltpu.async_copy(edges.at[0, blk, pl.ds(0, GCH)], src_idx.at[0], semi)
  pltpu.async_copy(edges.at[1, blk, pl.ds(0, GCH)], dst_idx.at[0], semi)

  _zero_fill(rows0, B, D)
  nfull = ROWS_PER_TILE // B
  tail = ROWS_PER_TILE - nfull * B
  for k in range(nfull):
    pltpu.async_copy(rows0, acc_sh.at[pl.ds(base + k * B, B)], semi)
  if tail:
    pltpu.async_copy(rows0.at[pl.ds(0, tail)],
                     acc_sh.at[pl.ds(base + nfull * B, tail)], semi)
  pltpu.make_async_copy(edges.at[0, blk, pl.ds(0, GCH)],
                        src_idx.at[0], semi).wait()
  pltpu.make_async_copy(edges.at[1, blk, pl.ds(0, GCH)],
                        dst_idx.at[0], semi).wait()
  for k in range(nfull):
    pltpu.make_async_copy(rows0, acc_sh.at[pl.ds(base, B)], semi).wait()
  if tail:
    pltpu.make_async_copy(rows0.at[pl.ds(0, tail)],
                          acc_sh.at[pl.ds(base, tail)], semi).wait()

  plsc.subcore_barrier()

  def start(p, t, buf, sem):
    pltpu.async_copy(feats.at[src_idx.at[p, t]], buf, sem)

  def wait(buf, sem):
    pltpu.make_async_copy(feats.at[pl.ds(0, B)], buf, sem).wait()

  def scat(p, t, buf):
    pltpu.sync_copy(buf, acc_sh.at[dst_idx.at[p, t]], add=True)

  start(0, 0, rows0, sem0)
  start(0, 1, rows1, sem1)

  for g in range(GROUPS):
    p = g % 2
    q = 1 - p
    if g < GROUPS - 1:
      pltpu.async_copy(edges.at[0, blk, pl.ds((g + 1) * GCH, GCH)],
                       src_idx.at[q], semi)
      pltpu.async_copy(edges.at[1, blk, pl.ds((g + 1) * GCH, GCH)],
                       dst_idx.at[q], semi)

    @pl.loop(0, GCH // 2 - 1)
    def _(i):
      t0 = 2 * i
      wait(rows0, sem0)
      scat(p, t0, rows0)
      start(p, t0 + 2, rows0, sem0)
      wait(rows1, sem1)
      scat(p, t0 + 1, rows1)
      start(p, t0 + 3, rows1, sem1)

    if g < GROUPS - 1:
      pltpu.make_async_copy(edges.at[0, blk, pl.ds(0, GCH)],
                            src_idx.at[q], semi).wait()
      pltpu.make_async_copy(edges.at[1, blk, pl.ds(0, GCH)],
                            dst_idx.at[q], semi).wait()
      wait(rows0, sem0)
      scat(p, GCH - 2, rows0)
      start(q, 0, rows0, sem0)
      wait(rows1, sem1)
      scat(p, GCH - 1, rows1)
      start(q, 1, rows1, sem1)
    else:
      wait(rows0, sem0)
      scat(p, GCH - 2, rows0)
      wait(rows1, sem1)
      scat(p, GCH - 1, rows1)

  plsc.subcore_barrier()

  pltpu.sync_copy(acc_sh.at[pl.ds(base, ROWS_PER_TILE)],
                  acc_out.at[cid, pl.ds(base, ROWS_PER_TILE)])


_sc_agg = pl.kernel(
    _sc_agg_body,
    out_type=[jax.ShapeDtypeStruct((NC, N, D), jnp.float32)],
    mesh=_MESH,
    scratch_types=[
        pltpu.VMEM((2, GCH, B), jnp.int32),
        pltpu.VMEM((2, GCH, B), jnp.int32),
        pltpu.VMEM((B, D), jnp.float32),
        pltpu.VMEM((B, D), jnp.float32),
        pltpu.SemaphoreType.DMA,
        pltpu.SemaphoreType.DMA,
        pltpu.SemaphoreType.DMA,
        pltpu.VMEM_SHARED((N, D), jnp.float32),
    ],
    compiler_params=_SC_PARAMS)


NB = N // 16
NBP = 640


def _sc_cnt_body(dst_h, cnt_out, dst_flat, bins, ident, cnt_sh):
  cid = lax.axis_index("c")
  sid = lax.axis_index("s")
  blk = cid * NS + sid

  pltpu.sync_copy(dst_h.at[pl.ds(blk * EPT, EPT)], dst_flat)
  _zero_fill(bins, NBP, 16)

  @pl.loop(0, 5)
  def _(r):
    for kk in range(8):
      ident[r, pl.ds(kk * 16, 16)] = (
          lax.broadcasted_iota(jnp.int32, (16,), 0) + r * 128 + kk * 16)

  @pl.when(sid == 0)
  def _():
    pltpu.sync_copy(bins, cnt_sh)

  plsc.subcore_barrier()

  ones16 = jnp.ones((16,), jnp.float32)

  @pl.loop(0, EPT // 16)
  def _(i):
    d = dst_flat[pl.ds(i * 16, 16)]
    row = lax.shift_right_logical(d, 4)
    col = lax.bitwise_and(d, 15)
    plsc.addupdate_scatter(bins, [row, col], ones16)

  for k in range(NBP // 128):
    pltpu.sync_copy(bins.at[pl.ds(k * 128, 128)],
                    cnt_sh.at[ident.at[k]], add=True)

  plsc.subcore_barrier()

  @pl.when(sid == 0)
  def _():
    pltpu.sync_copy(cnt_sh, cnt_out.at[cid])


_sc_cnt = pl.kernel(
    _sc_cnt_body,
    out_type=[jax.ShapeDtypeStruct((NC, NBP, 16), jnp.float32)],
    mesh=_MESH,
    scratch_types=[
        pltpu.VMEM((EPT,), jnp.int32),
        pltpu.VMEM((NBP, 16), jnp.float32),
        pltpu.VMEM((5, 128), jnp.int32),
        pltpu.VMEM_SHARED((NBP, 16), jnp.float32),
    ],
    compiler_params=_SC_PARAMS)


_RB = 2000
_GRID = N // _RB


def _dot(a, b):
  return lax.dot_general(a, b, (((1,), (0,)), ((), ())),
                         precision=lax.Precision.DEFAULT,
                         preferred_element_type=jnp.float32)


def _mm_body(x_ref, w_ref, o_ref):
  o_ref[...] = _dot(x_ref[...], w_ref[...])


@jax.jit
def _mm(x, w):
  return pl.pallas_call(
      _mm_body,
      grid=(_GRID,),
      in_specs=[
          pl.BlockSpec((_RB, D), lambda i: (i, 0)),
          pl.BlockSpec((D, D), lambda i: (0, 0)),
      ],
      out_specs=pl.BlockSpec((_RB, D), lambda i: (i, 0)),
      out_shape=jax.ShapeDtypeStruct((N, D), jnp.float32),
  )(x, w)


def _rcnt_block(cnt_ref):
  i = pl.program_id(0)
  c = cnt_ref[0, i] + cnt_ref[1, i]
  return 1.0 / jnp.maximum(c, 1.0)


def _mid_body(acc_ref, cnt_ref, xr_ref, b1_ref, wl_ref, wr_ref,
              ol_ref, or_ref):
  s = acc_ref[0] + acc_ref[1]
  rc = _rcnt_block(cnt_ref)
  h = jnp.maximum(s * rc[:, None] + b1_ref[...] + xr_ref[...], 0.0)
  ol_ref[...] = _dot(h, wl_ref[...])
  or_ref[...] = _dot(h, wr_ref[...])


@jax.jit
def _mid(acc, cnt, xr, b1, wl, wr):
  return pl.pallas_call(
      _mid_body,
      grid=(_GRID,),
      in_specs=[
          pl.BlockSpec((NC, _RB, D), lambda i: (0, i, 0)),
          pl.BlockSpec((NC, _GRID, _RB), lambda i: (0, 0, 0)),
          pl.BlockSpec((_RB, D), lambda i: (i, 0)),
          pl.BlockSpec((1, D), lambda i: (0, 0)),
          pl.BlockSpec((D, D), lambda i: (0, 0)),
          pl.BlockSpec((D, D), lambda i: (0, 0)),
      ],
      out_specs=[
          pl.BlockSpec((_RB, D), lambda i: (i, 0)),
          pl.BlockSpec((_RB, D), lambda i: (i, 0)),
      ],
      out_shape=[
          jax.ShapeDtypeStruct((N, D), jnp.float32),
          jax.ShapeDtypeStruct((N, D), jnp.float32),
      ],
  )(acc, cnt, xr, b1, wl, wr)


def _final_body(acc_ref, cnt_ref, hr_ref, b2_ref, o_ref):
  s = acc_ref[0] + acc_ref[1]
  rc = _rcnt_block(cnt_ref)
  o_ref[...] = s * rc[:, None] + b2_ref[...] + hr_ref[...]


@jax.jit
def _final(acc, cnt, hr, b2):
  return pl.pallas_call(
      _final_body,
      grid=(_GRID,),
      in_specs=[
          pl.BlockSpec((NC, _RB, D), lambda i: (0, i, 0)),
          pl.BlockSpec((NC, _GRID, _RB), lambda i: (0, 0, 0)),
          pl.BlockSpec((_RB, D), lambda i: (i, 0)),
          pl.BlockSpec((1, D), lambda i: (0, 0)),
      ],
      out_specs=pl.BlockSpec((_RB, D), lambda i: (i, 0)),
      out_shape=jax.ShapeDtypeStruct((N, D), jnp.float32),
  )(acc, cnt, hr, b2)


@jax.jit
def kernel(x, edge_index, W1_l, b1_l, W1_r, W2_l, b2_l, W2_r):
  edges = edge_index.reshape(2, NW, CHUNKS, B)
  dst = edge_index[1]
  xl = _mm(x, W1_l)
  xr = _mm(x, W1_r)
  cnt, = _sc_cnt(dst)
  cnt = cnt.reshape(NC, NBP * 16)[:, :N].reshape(NC, _GRID, _RB)
  acc1, = _sc_agg(xl, edges)
  h2l, h2r = _mid(acc1, cnt, xr, b1_l.reshape(1, D), W2_l, W2_r)
  acc2, = _sc_agg(h2l, edges)
  return _final(acc2, cnt, h2r, b2_l.reshape(1, D))


# speedup vs baseline: 1.2081x; 1.0414x over previous
import functools

import jax
import jax.numpy as jnp
from jax import lax
from jax.experimental import pallas as pl
from jax.experimental.pallas import tpu as pltpu
from jax.experimental.pallas import tpu_sc as plsc

N = 10000
E = 320000
D = 128

NC = 2
NS = 16
NW = NC * NS
EPT = E // NW
ROWS_PER_TILE = N // NS

B = 128
EROWS = E // B
CHUNKS = EROWS // NW
EXTRA = EROWS - CHUNKS * NW
GROUPS = 3
GCH = CHUNKS // GROUPS


def _zero_fill(ref, nrows, ncols):
  @pl.loop(0, nrows)
  def _(r):
    for k in range(ncols // 16):
      ref[r, pl.ds(16 * k, 16)] = jnp.zeros((16,), jnp.float32)


_MESH = plsc.VectorSubcoreMesh(core_axis_name="c", subcore_axis_name="s")
_SC_PARAMS = pltpu.CompilerParams(use_tc_tiling_on_sc=False,
                                  needs_layout_passes=False)


def _sc_agg_body(feats, edges, acc_out,
                 src_idx, dst_idx, xsrc, xdst, rows0, rows1,
                 sem0, sem1, semi, acc_sh):
  cid = lax.axis_index("c")
  sid = lax.axis_index("s")
  blk = cid * NS + sid
  base = sid * ROWS_PER_TILE
  erow = blk * CHUNKS

  pltpu.async_copy(edges.at[0, pl.ds(erow, GCH)], src_idx.at[0], semi)
  pltpu.async_copy(edges.at[1, pl.ds(erow, GCH)], dst_idx.at[0], semi)

  @pl.when(blk < EXTRA)
  def _():
    pltpu.async_copy(edges.at[0, pl.ds(NW * CHUNKS + blk, 1)], xsrc, semi)
    pltpu.async_copy(edges.at[1, pl.ds(NW * CHUNKS + blk, 1)], xdst, semi)

  _zero_fill(rows0, B, D)
  nfull = ROWS_PER_TILE // B
  tail = ROWS_PER_TILE - nfull * B
  for k in range(nfull):
    pltpu.async_copy(rows0, acc_sh.at[pl.ds(base + k * B, B)], semi)
  if tail:
    pltpu.async_copy(rows0.at[pl.ds(0, tail)],
                     acc_sh.at[pl.ds(base + nfull * B, tail)], semi)
  pltpu.make_async_copy(edges.at[0, pl.ds(0, GCH)],
                        src_idx.at[0], semi).wait()
  pltpu.make_async_copy(edges.at[1, pl.ds(0, GCH)],
                        dst_idx.at[0], semi).wait()
  @pl.when(blk < EXTRA)
  def _():
    pltpu.make_async_copy(edges.at[0, pl.ds(0, 1)], xsrc, semi).wait()
    pltpu.make_async_copy(edges.at[1, pl.ds(0, 1)], xdst, semi).wait()
  for k in range(nfull):
    pltpu.make_async_copy(rows0, acc_sh.at[pl.ds(base, B)], semi).wait()
  if tail:
    pltpu.make_async_copy(rows0.at[pl.ds(0, tail)],
                          acc_sh.at[pl.ds(base, tail)], semi).wait()

  plsc.subcore_barrier()

  def start(p, t, buf, sem):
    pltpu.async_copy(feats.at[src_idx.at[p, t]], buf, sem)

  def wait(buf, sem):
    pltpu.make_async_copy(feats.at[pl.ds(0, B)], buf, sem).wait()

  def scat(p, t, buf):
    pltpu.sync_copy(buf, acc_sh.at[dst_idx.at[p, t]], add=True)

  start(0, 0, rows0, sem0)
  start(0, 1, rows1, sem1)

  for g in range(GROUPS):
    p = g % 2
    q = 1 - p
    if g < GROUPS - 1:
      pltpu.async_copy(edges.at[0, pl.ds(erow + (g + 1) * GCH, GCH)],
                       src_idx.at[q], semi)
      pltpu.async_copy(edges.at[1, pl.ds(erow + (g + 1) * GCH, GCH)],
                       dst_idx.at[q], semi)

    @pl.loop(0, GCH // 2 - 1)
    def _(i):
      t0 = 2 * i
      wait(rows0, sem0)
      scat(p, t0, rows0)
      start(p, t0 + 2, rows0, sem0)
      wait(rows1, sem1)
      scat(p, t0 + 1, rows1)
      start(p, t0 + 3, rows1, sem1)

    if g < GROUPS - 1:
      pltpu.make_async_copy(edges.at[0, pl.ds(0, GCH)],
                            src_idx.at[q], semi).wait()
      pltpu.make_async_copy(edges.at[1, pl.ds(0, GCH)],
                            dst_idx.at[q], semi).wait()
      wait(rows0, sem0)
      scat(p, GCH - 2, rows0)
      start(q, 0, rows0, sem0)
      wait(rows1, sem1)
      scat(p, GCH - 1, rows1)
      start(q, 1, rows1, sem1)
    else:
      wait(rows0, sem0)
      scat(p, GCH - 2, rows0)
      wait(rows1, sem1)
      scat(p, GCH - 1, rows1)

  @pl.when(blk < EXTRA)
  def _():
    pltpu.async_copy(feats.at[xsrc.at[0]], rows0, sem0)
    pltpu.make_async_copy(feats.at[pl.ds(0, B)], rows0, sem0).wait()
    pltpu.sync_copy(rows0, acc_sh.at[xdst.at[0]], add=True)

  plsc.subcore_barrier()

  pltpu.sync_copy(acc_sh.at[pl.ds(base, ROWS_PER_TILE)],
                  acc_out.at[cid, pl.ds(base, ROWS_PER_TILE)])


_sc_agg = pl.kernel(
    _sc_agg_body,
    out_type=[jax.ShapeDtypeStruct((NC, N, D), jnp.float32)],
    mesh=_MESH,
    scratch_types=[
        pltpu.VMEM((2, GCH, B), jnp.int32),
        pltpu.VMEM((2, GCH, B), jnp.int32),
        pltpu.VMEM((1, B), jnp.int32),
        pltpu.VMEM((1, B), jnp.int32),
        pltpu.VMEM((B, D), jnp.float32),
        pltpu.VMEM((B, D), jnp.float32),
        pltpu.SemaphoreType.DMA,
        pltpu.SemaphoreType.DMA,
        pltpu.SemaphoreType.DMA,
        pltpu.VMEM_SHARED((N, D), jnp.float32),
    ],
    compiler_params=_SC_PARAMS)


NB = N // 16
NBP = 640


def _sc_cnt_body(edges, cnt_out, dst2d, xdst, bins, ident, cnt_sh):
  cid = lax.axis_index("c")
  sid = lax.axis_index("s")
  blk = cid * NS + sid

  pltpu.sync_copy(edges.at[1, pl.ds(blk * CHUNKS, CHUNKS)], dst2d)
  @pl.when(blk < EXTRA)
  def _():
    pltpu.sync_copy(edges.at[1, pl.ds(NW * CHUNKS + blk, 1)], xdst)
  _zero_fill(bins, NBP, 16)

  @pl.loop(0, 5)
  def _(r):
    for kk in range(8):
      ident[r, pl.ds(kk * 16, 16)] = (
          lax.broadcasted_iota(jnp.int32, (16,), 0) + r * 128 + kk * 16)

  @pl.when(sid == 0)
  def _():
    pltpu.sync_copy(bins, cnt_sh)

  plsc.subcore_barrier()

  ones16 = jnp.ones((16,), jnp.float32)

  def hist16(d):
    row = lax.shift_right_logical(d, 4)
    col = lax.bitwise_and(d, 15)
    plsc.addupdate_scatter(bins, [row, col], ones16)

  @pl.loop(0, CHUNKS)
  def _(i):
    for c in range(B // 16):
      hist16(dst2d[i, pl.ds(c * 16, 16)])

  @pl.when(blk < EXTRA)
  def _():
    for c in range(B // 16):
      hist16(xdst[0, pl.ds(c * 16, 16)])

  for k in range(NBP // 128):
    pltpu.sync_copy(bins.at[pl.ds(k * 128, 128)],
                    cnt_sh.at[ident.at[k]], add=True)

  plsc.subcore_barrier()

  @pl.when(sid == 0)
  def _():
    pltpu.sync_copy(cnt_sh, cnt_out.at[cid])


_sc_cnt = pl.kernel(
    _sc_cnt_body,
    out_type=[jax.ShapeDtypeStruct((NC, NBP, 16), jnp.float32)],
    mesh=_MESH,
    scratch_types=[
        pltpu.VMEM((CHUNKS, B), jnp.int32),
        pltpu.VMEM((1, B), jnp.int32),
        pltpu.VMEM((NBP, 16), jnp.float32),
        pltpu.VMEM((5, 128), jnp.int32),
        pltpu.VMEM_SHARED((NBP, 16), jnp.float32),
    ],
    compiler_params=_SC_PARAMS)


_RB = 2000
_GRID = N // _RB


def _dot(a, b):
  return lax.dot_general(a, b, (((1,), (0,)), ((), ())),
                         precision=lax.Precision.DEFAULT,
                         preferred_element_type=jnp.float32)


def _mm_body(x_ref, w_ref, o_ref):
  o_ref[...] = _dot(x_ref[...], w_ref[...])


@jax.jit
def _mm(x, w):
  return pl.pallas_call(
      _mm_body,
      grid=(_GRID,),
      in_specs=[
          pl.BlockSpec((_RB, D), lambda i: (i, 0)),
          pl.BlockSpec((D, D), lambda i: (0, 0)),
      ],
      out_specs=pl.BlockSpec((_RB, D), lambda i: (i, 0)),
      out_shape=jax.ShapeDtypeStruct((N, D), jnp.float32),
  )(x, w)


def _rcnt_block(cnt_ref):
  i = pl.program_id(0)
  c = cnt_ref[0, i] + cnt_ref[1, i]
  return 1.0 / jnp.maximum(c, 1.0)


def _mid_body(acc_ref, cnt_ref, xr_ref, b1_ref, wl_ref, wr_ref,
              ol_ref, or_ref):
  s = acc_ref[0] + acc_ref[1]
  rc = _rcnt_block(cnt_ref)
  h = jnp.maximum(s * rc[:, None] + b1_ref[...] + xr_ref[...], 0.0)
  ol_ref[...] = _dot(h, wl_ref[...])
  or_ref[...] = _dot(h, wr_ref[...])


@jax.jit
def _mid(acc, cnt, xr, b1, wl, wr):
  return pl.pallas_call(
      _mid_body,
      grid=(_GRID,),
      in_specs=[
          pl.BlockSpec((NC, _RB, D), lambda i: (0, i, 0)),
          pl.BlockSpec((NC, _GRID, _RB), lambda i: (0, 0, 0)),
          pl.BlockSpec((_RB, D), lambda i: (i, 0)),
          pl.BlockSpec((1, D), lambda i: (0, 0)),
          pl.BlockSpec((D, D), lambda i: (0, 0)),
          pl.BlockSpec((D, D), lambda i: (0, 0)),
      ],
      out_specs=[
          pl.BlockSpec((_RB, D), lambda i: (i, 0)),
          pl.BlockSpec((_RB, D), lambda i: (i, 0)),
      ],
      out_shape=[
          jax.ShapeDtypeStruct((N, D), jnp.float32),
          jax.ShapeDtypeStruct((N, D), jnp.float32),
      ],
  )(acc, cnt, xr, b1, wl, wr)


def _final_body(acc_ref, cnt_ref, hr_ref, b2_ref, o_ref):
  s = acc_ref[0] + acc_ref[1]
  rc = _rcnt_block(cnt_ref)
  o_ref[...] = s * rc[:, None] + b2_ref[...] + hr_ref[...]


@jax.jit
def _final(acc, cnt, hr, b2):
  return pl.pallas_call(
      _final_body,
      grid=(_GRID,),
      in_specs=[
          pl.BlockSpec((NC, _RB, D), lambda i: (0, i, 0)),
          pl.BlockSpec((NC, _GRID, _RB), lambda i: (0, 0, 0)),
          pl.BlockSpec((_RB, D), lambda i: (i, 0)),
          pl.BlockSpec((1, D), lambda i: (0, 0)),
      ],
      out_specs=pl.BlockSpec((_RB, D), lambda i: (i, 0)),
      out_shape=jax.ShapeDtypeStruct((N, D), jnp.float32),
  )(acc, cnt, hr, b2)


@jax.jit
def kernel(x, edge_index, W1_l, b1_l, W1_r, W2_l, b2_l, W2_r):
  edges = edge_index.reshape(2, EROWS, B)
  xl = _mm(x, W1_l)
  xr = _mm(x, W1_r)
  cnt, = _sc_cnt(edges)
  cnt = cnt.reshape(NC, NBP * 16)[:, :N].reshape(NC, _GRID, _RB)
  acc1, = _sc_agg(xl, edges)
  h2l, h2r = _mid(acc1, cnt, xr, b1_l.reshape(1, D), W2_l, W2_r)
  acc2, = _sc_agg(h2l, edges)
  return _final(acc2, cnt, h2r, b2_l.reshape(1, D))
